# baseline (speedup 1.0000x reference)
import numpy as np
import jax
import jax.numpy as jnp
from jax import lax
from jax.experimental import pallas as pl
from jax.experimental.pallas import tpu as pltpu
from jax.experimental.pallas import tpu_sc as plsc

_B = 4
_V = 20170
_EMB = 64
_N = _B * _V
_NF = 10
_TDIM = 64

_NC = 2
_NS = 16
_NW = _NC * _NS
_CH = 128
_NCH = 20
_RPW = _CH * _NCH
_NPAD = _NW * _RPW

_R = 40
_GRID = _N // _R


def _sc_gather_body(idx_l, idx_r, tab_l, tab_r, out_l, out_r,
                    idxl_v, idxr_v, rows_l, rows_r, sem_l, sem_r):
    wid = lax.axis_index("s") * _NC + lax.axis_index("c")
    base = wid * _RPW
    pltpu.sync_copy(idx_l.at[wid], idxl_v)
    pltpu.sync_copy(idx_r.at[wid], idxr_v)

    def chunk(j, carry):
        cl = pltpu.async_copy(tab_l.at[idxl_v.at[j]], rows_l, sem_l)
        cr = pltpu.async_copy(tab_r.at[idxr_v.at[j]], rows_r, sem_r)
        cl.wait()
        pltpu.sync_copy(rows_l, out_l.at[pl.ds(base + j * _CH, _CH)])
        cr.wait()
        pltpu.sync_copy(rows_r, out_r.at[pl.ds(base + j * _CH, _CH)])
        return carry

    lax.fori_loop(0, _NCH, chunk, 0)


def _make_sc_gather():
    return pl.kernel(
        _sc_gather_body,
        out_type=(
            jax.ShapeDtypeStruct((_NPAD, _EMB), jnp.float32),
            jax.ShapeDtypeStruct((_NPAD, _EMB), jnp.float32),
        ),
        mesh=plsc.VectorSubcoreMesh(core_axis_name="c", subcore_axis_name="s"),
        scratch_types=(
            pltpu.VMEM((_NCH, _CH), jnp.int32),
            pltpu.VMEM((_NCH, _CH), jnp.int32),
            pltpu.VMEM((_CH, _EMB), jnp.float32),
            pltpu.VMEM((_CH, _EMB), jnp.float32),
            pltpu.SemaphoreType.DMA,
            pltpu.SemaphoreType.DMA,
        ),
        compiler_params=pltpu.CompilerParams(use_tc_tiling_on_sc=False),
    )


def _pe_tables():
    f = np.zeros(63, np.float32)
    ax = np.zeros(63, np.int64)
    raw = np.zeros(63, bool)
    is_sin = np.zeros(63, bool)
    for c in range(3):
        f[c], ax[c], raw[c] = 1.0, c, True
    for k in range(60):
        c = 3 + k
        j, m = k // 6, k % 6
        ax[c] = m % 3
        f[c] = np.float32(2.0 ** j)
        is_sin[c] = m < 3
    sel_l = [np.where((ax == i) & np.ones(63, bool), f, 0.0) for i in range(3)]
    sl = np.zeros((3, 126), np.float32)
    sr = np.zeros((3, 126), np.float32)
    for i in range(3):
        sl[i, :63] = sel_l[i]
        sr[i, 63:] = sel_l[i]
    raw2 = np.concatenate([raw, raw])
    sin2 = np.concatenate([is_sin, is_sin])
    return sl, sr, raw2, sin2


_SL_np, _SR_np, _RAW_np, _SIN_np = _pe_tables()
_PC_np = np.concatenate(
    [_SL_np, _SR_np,
     _RAW_np.astype(np.float32).reshape(1, 126),
     _SIN_np.astype(np.float32).reshape(1, 126)], axis=0)
_TEF_np = np.exp(np.arange(_TDIM // 2, dtype=np.float32)
                 * (-np.log(10000.0) / (_TDIM // 2 - 1))).reshape(1, -1)


def _tc_body(emb_l_ref, emb_r_ref, g_ref, t_ref, wb_ref, pc_ref, tef_ref,
             out_ref, temb_ref):
    w = wb_ref[0:1, :]
    bvec = wb_ref[1:2, :]
    out_ref[:, 0:64] = emb_l_ref[...]
    out_ref[:, 64:128] = g_ref[:, 0:1] * w + bvec
    out_ref[:, 128:192] = emb_r_ref[...]
    out_ref[:, 192:256] = g_ref[:, 4:5] * w + bvec

    raw = pc_ref[6:7, :] > 0.5
    sinm = pc_ref[7:8, :] > 0.5
    a = (g_ref[:, 1:2] * pc_ref[0:1, :] + g_ref[:, 2:3] * pc_ref[1:2, :]
         + g_ref[:, 3:4] * pc_ref[2:3, :]
         + g_ref[:, 5:6] * pc_ref[3:4, :] + g_ref[:, 6:7] * pc_ref[4:5, :]
         + g_ref[:, 7:8] * pc_ref[5:6, :])
    out_ref[:, 256:382] = jnp.where(raw, a,
                                    jnp.where(sinm, jnp.sin(a), jnp.cos(a)))

    @pl.when(pl.program_id(0) == 0)
    def _():
        te = t_ref[...] * tef_ref[...]
        temb_ref[:, 0:32] = jnp.sin(te)
        temb_ref[:, 32:64] = jnp.cos(te)


_tc_assemble = pl.pallas_call(
    _tc_body,
    grid=(_GRID,),
    in_specs=[
        pl.BlockSpec((_R, _EMB), lambda i: (i, 0)),
        pl.BlockSpec((_R, _EMB), lambda i: (i, 0)),
        pl.BlockSpec((_R, 8), lambda i: (i, 0)),
        pl.BlockSpec((_B, 1), lambda i: (0, 0)),
        pl.BlockSpec((2, 64), lambda i: (0, 0)),
        pl.BlockSpec((8, 126), lambda i: (0, 0)),
        pl.BlockSpec((1, 32), lambda i: (0, 0)),
    ],
    out_specs=[
        pl.BlockSpec((_R, 382), lambda i: (i, 0)),
        pl.BlockSpec((_B, _TDIM), lambda i: (0, 0)),
    ],
    out_shape=[
        jax.ShapeDtypeStruct((_N, 382), jnp.float32),
        jax.ShapeDtypeStruct((_B, _TDIM), jnp.float32),
    ],
)


def kernel(idx_left, idx_right, gripper_open_left, gripper_open_right,
           gripper_pos_left, gripper_pos_right, t,
           table_left, table_right, W_proj, b_proj):
    idx_l = jnp.pad(idx_left.reshape(_N), (0, _NPAD - _N)).reshape(_NW, _NCH, _CH)
    idx_r = jnp.pad(idx_right.reshape(_N), (0, _NPAD - _N)).reshape(_NW, _NCH, _CH)
    emb_l, emb_r = _make_sc_gather()(idx_l, idx_r, table_left, table_right)

    g = jnp.concatenate(
        [gripper_open_left, gripper_pos_left,
         gripper_open_right, gripper_pos_right], axis=-1).reshape(_N, 8)
    t_f = t.astype(jnp.float32).reshape(_B, 1)
    wb = jnp.concatenate([W_proj.reshape(1, _EMB), b_proj.reshape(1, _EMB)], 0)

    out_flat, temb = _tc_assemble(emb_l, emb_r, g, t_f, wb,
                                  jnp.asarray(_PC_np), jnp.asarray(_TEF_np))
    return out_flat.reshape(_B, _V, 382), temb
